# baseline (speedup 1.0000x reference)
import functools

import jax
import jax.numpy as jnp
from jax import lax
from jax.experimental import pallas as pl
from jax.experimental.pallas import tpu as pltpu
from jax.experimental.pallas import tpu_sc as plsc


def _sc_gather_body(per_worker, table_hbm, idx_hbm, out_hbm,
                    idx_v, rows_v, sem):
    info = plsc.get_sparse_core_info()
    nc = info.num_cores
    wid = lax.axis_index("s") * nc + lax.axis_index("c")
    base = wid * per_worker
    pltpu.sync_copy(idx_hbm.at[pl.ds(base, per_worker)], idx_v)
    pltpu.async_copy(table_hbm.at[idx_v], rows_v, sem).wait()
    pltpu.sync_copy(rows_v, out_hbm.at[pl.ds(base, per_worker)])


def _sc_gather(table, idx):
    n_tokens, emb = idx.shape[0], table.shape[1]
    info = plsc.get_sparse_core_info()
    n_workers = info.num_cores * info.num_subcores
    assert n_tokens % (8 * n_workers) == 0
    per_worker = n_tokens // n_workers
    mesh = plsc.VectorSubcoreMesh(core_axis_name="c", subcore_axis_name="s")
    body = functools.partial(_sc_gather_body, per_worker)
    return pl.kernel(
        body,
        out_type=jax.ShapeDtypeStruct((n_tokens, emb), jnp.float32),
        mesh=mesh,
        scratch_types=[
            pltpu.VMEM((per_worker,), jnp.int32),
            pltpu.VMEM((per_worker, emb), jnp.float32),
            pltpu.SemaphoreType.DMA,
        ],
    )(table, idx)


def _proj_body(t_ref, pos_ref, w_ref, out_ref, h_ref):
    @pl.when(pl.program_id(0) == 0)
    def _():
        reps = t_ref.shape[0] // pos_ref.shape[0]
        p = jnp.concatenate([pos_ref[...]] * reps, axis=0)
        h_ref[...] = t_ref[...] + p

    out_ref[...] = lax.dot_general(
        h_ref[...], w_ref[...],
        dimension_numbers=(((1,), (1,)), ((), ())),
        preferred_element_type=jnp.float32,
    )


def _projection(t, pos, w_out, block_n):
    m, emb = t.shape
    vocab = w_out.shape[0]
    grid = (pl.cdiv(vocab, block_n),)
    return pl.pallas_call(
        _proj_body,
        grid=grid,
        in_specs=[
            pl.BlockSpec((m, emb), lambda j: (0, 0)),
            pl.BlockSpec(pos.shape, lambda j: (0, 0)),
            pl.BlockSpec((block_n, emb), lambda j: (j, 0)),
        ],
        out_specs=pl.BlockSpec((m, block_n), lambda j: (0, j)),
        out_shape=jax.ShapeDtypeStruct((m, vocab), jnp.float32),
        scratch_shapes=[pltpu.VMEM((m, emb), jnp.float32)],
    )(t, pos, w_out)


def kernel(x, tok_emb, pos_emb, W_out):
    b, s = x.shape
    idx = x.reshape(-1).astype(jnp.int32)
    t = _sc_gather(tok_emb, idx)
    logits = _projection(t, pos_emb, W_out, block_n=512)
    return logits.reshape(b, s, -1)
